# baseline (speedup 1.0000x reference)
import jax
import jax.numpy as jnp
from jax import lax
from jax.experimental import pallas as pl
from jax.experimental.pallas import tpu as pltpu
from jax.experimental.pallas import tpu_sc as plsc

B, F, D = 16384, 100, 64
NC, NS, L = 2, 16, 16
NW = NC * NS
BPW = B // NW
RING = 4


def _sc_body(x_hbm, emb_hbm, w_hbm, bias_hbm, out_hbm,
             basebuf, wvbuf, biasbuf,
             xb0, xb1, ob0, ob1, ob2, ob3,
             sx0, sx1, so0, so1, so2, so3):
    wid = lax.axis_index("s") * NC + lax.axis_index("c")
    x0 = wid * (BPW * F)
    b0 = wid * BPW

    pltpu.sync_copy(emb_hbm, basebuf)
    pltpu.sync_copy(w_hbm, wvbuf)
    pltpu.sync_copy(bias_hbm, biasbuf)

    bias_regs = [biasbuf[pl.ds(dc * L, L)] for dc in range(4)]
    wv_regs = [wvbuf[pl.ds(dc * L, L)] for dc in range(4)]

    @plsc.parallel_loop(0, F)
    def _fold(f):
        for dc in range(4):
            o = f * D + dc * L
            basebuf[pl.ds(o, L)] = basebuf[pl.ds(o, L)] + bias_regs[dc]

    xbufs = (xb0, xb1)
    obufs = (ob0, ob1, ob2, ob3)
    sxs = (sx0, sx1)
    sos = (so0, so1, so2, so3)

    pltpu.async_copy(x_hbm.at[pl.ds(x0, 2 * F)], xb0.at[pl.ds(8, 2 * F)], sx0)
    pltpu.async_copy(
        x_hbm.at[pl.ds(x0 + 2 * F, 2 * F)], xb1.at[pl.ds(8, 2 * F)], sx1)

    @pl.loop(0, BPW, step=RING)
    def _quad(j):
        for k in range(RING):
            n = j + k
            xb, sx = xbufs[k // 2], sxs[k // 2]
            ob, so = obufs[k], sos[k]
            if k % 2 == 0:
                pltpu.make_async_copy(
                    x_hbm.at[pl.ds(x0, 2 * F)],
                    xb.at[pl.ds(8, 2 * F)], sx).wait()

            @pl.when(n >= RING)
            def _():
                pltpu.make_async_copy(
                    ob, out_hbm.at[pl.ds(b0, 1)], so).wait()

            off = 8 + (k % 2) * F
            for f in range(F):
                xs = plsc.load_gather(
                    xb, [jnp.full((L,), off + f, jnp.int32)])
                for dc in range(4):
                    ob[0, f, pl.ds(dc * L, L)] = (
                        xs * wv_regs[dc]
                        + basebuf[pl.ds(f * D + dc * L, L)])

            pltpu.async_copy(ob, out_hbm.at[pl.ds(b0 + n, 1)], so)

            if k % 2 == 1:
                @pl.when(n + RING < BPW)
                def _():
                    pltpu.async_copy(
                        x_hbm.at[pl.ds(x0 + (n + 3) * F, 2 * F)],
                        xb.at[pl.ds(8, 2 * F)], sx)

    for k in range(RING):
        pltpu.make_async_copy(
            obufs[k], out_hbm.at[pl.ds(b0, 1)], sos[k]).wait()


@jax.jit
def kernel(x, emb_table, W, b):
    mesh = plsc.VectorSubcoreMesh(
        core_axis_name="c", subcore_axis_name="s",
        num_cores=NC, num_subcores=NS)
    return pl.kernel(
        _sc_body,
        out_type=jax.ShapeDtypeStruct((B, F, D), jnp.float32),
        mesh=mesh,
        scratch_types=(
            [
                pltpu.VMEM((F * D,), jnp.float32),
                pltpu.VMEM((D,), jnp.float32),
                pltpu.VMEM((D,), jnp.float32),
            ]
            + [pltpu.VMEM((8 + 2 * F,), jnp.float32) for _ in range(2)]
            + [pltpu.VMEM((1, F, D), jnp.float32) for _ in range(RING)]
            + [pltpu.SemaphoreType.DMA for _ in range(2 + RING)]
        ),
        compiler_params=pltpu.CompilerParams(needs_layout_passes=False),
    )(x.reshape(-1), emb_table.reshape(-1), W.reshape(-1), b)


# speedup vs baseline: 1.8601x; 1.8601x over previous
import jax
import jax.numpy as jnp
from jax import lax
from jax.experimental import pallas as pl
from jax.experimental.pallas import tpu as pltpu
from jax.experimental.pallas import tpu_sc as plsc

B, F, D = 16384, 100, 64
NC, NS, L = 2, 16, 16
NW = NC * NS
BPW = B // NW
RING = 4


def _sc_body(x_hbm, emb_hbm, w_hbm, bias_hbm, out_hbm,
             basebuf, wvbuf, biasbuf,
             xb0, xb1, ob0, ob1, ob2, ob3,
             sx0, sx1, so0, so1, so2, so3):
    wid = lax.axis_index("s") * NC + lax.axis_index("c")
    x0 = wid * (BPW * F)
    b0 = wid * BPW

    pltpu.sync_copy(emb_hbm, basebuf)
    pltpu.sync_copy(w_hbm, wvbuf)
    pltpu.sync_copy(bias_hbm, biasbuf)

    bias_regs = [biasbuf[pl.ds(dc * L, L)] for dc in range(4)]
    wv_regs = [wvbuf[pl.ds(dc * L, L)] for dc in range(4)]

    @plsc.parallel_loop(0, F)
    def _fold(f):
        for dc in range(4):
            o = f * D + dc * L
            basebuf[pl.ds(o, L)] = basebuf[pl.ds(o, L)] + bias_regs[dc]

    xbufs = (xb0, xb1)
    obufs = (ob0, ob1, ob2, ob3)
    sxs = (sx0, sx1)
    sos = (so0, so1, so2, so3)

    pltpu.async_copy(x_hbm.at[pl.ds(x0, 2 * F)], xb0.at[pl.ds(8, 2 * F)], sx0)
    pltpu.async_copy(
        x_hbm.at[pl.ds(x0 + 2 * F, 2 * F)], xb1.at[pl.ds(8, 2 * F)], sx1)

    @pl.loop(0, BPW, step=RING)
    def _quad(j):
        for k in range(RING):
            n = j + k
            xb, sx = xbufs[k // 2], sxs[k // 2]
            ob, so = obufs[k], sos[k]
            if k % 2 == 0:
                pltpu.make_async_copy(
                    x_hbm.at[pl.ds(x0, 2 * F)],
                    xb.at[pl.ds(8, 2 * F)], sx).wait()

            @pl.when(n >= RING)
            def _():
                pltpu.make_async_copy(
                    ob, out_hbm.at[pl.ds(b0, 1)], so).wait()

            off = 8 + (k % 2) * F

            @plsc.parallel_loop(0, F, unroll=2)
            def _row(f):
                xs = plsc.load_gather(
                    xb, [jnp.full((L,), off + f, jnp.int32)])
                for dc in range(4):
                    ob[0, f, pl.ds(dc * L, L)] = (
                        xs * wv_regs[dc]
                        + basebuf[pl.ds(f * D + dc * L, L)])

            pltpu.async_copy(ob, out_hbm.at[pl.ds(b0 + n, 1)], so)

            if k % 2 == 1:
                @pl.when(n + RING < BPW)
                def _():
                    pltpu.async_copy(
                        x_hbm.at[pl.ds(x0 + (n + 3) * F, 2 * F)],
                        xb.at[pl.ds(8, 2 * F)], sx)

    for k in range(RING):
        pltpu.make_async_copy(
            obufs[k], out_hbm.at[pl.ds(b0, 1)], sos[k]).wait()


@jax.jit
def kernel(x, emb_table, W, b):
    mesh = plsc.VectorSubcoreMesh(
        core_axis_name="c", subcore_axis_name="s",
        num_cores=NC, num_subcores=NS)
    return pl.kernel(
        _sc_body,
        out_type=jax.ShapeDtypeStruct((B, F, D), jnp.float32),
        mesh=mesh,
        scratch_types=(
            [
                pltpu.VMEM((F * D,), jnp.float32),
                pltpu.VMEM((D,), jnp.float32),
                pltpu.VMEM((D,), jnp.float32),
            ]
            + [pltpu.VMEM((8 + 2 * F,), jnp.float32) for _ in range(2)]
            + [pltpu.VMEM((1, F, D), jnp.float32) for _ in range(RING)]
            + [pltpu.SemaphoreType.DMA for _ in range(2 + RING)]
        ),
        compiler_params=pltpu.CompilerParams(needs_layout_passes=False),
    )(x.reshape(-1), emb_table.reshape(-1), W.reshape(-1), b)


# speedup vs baseline: 1.8653x; 1.0028x over previous
import jax
import jax.numpy as jnp
from jax import lax
from jax.experimental import pallas as pl
from jax.experimental.pallas import tpu as pltpu
from jax.experimental.pallas import tpu_sc as plsc

B, F, D = 16384, 100, 64
NC, NS, L = 2, 16, 16
NW = NC * NS
BPW = B // NW
CBB = 4
XW = CBB * F


def _sc_body(x_hbm, emb_hbm, w_hbm, bias_hbm, out_hbm,
             basebuf, wvbuf, biasbuf,
             xb0, xb1, ob0, ob1,
             sx0, sx1, so0, so1):
    wid = lax.axis_index("s") * NC + lax.axis_index("c")
    x0 = wid * (BPW * F)
    b0 = wid * BPW

    pltpu.sync_copy(emb_hbm, basebuf)
    pltpu.sync_copy(w_hbm, wvbuf)
    pltpu.sync_copy(bias_hbm, biasbuf)

    bias_regs = [biasbuf[pl.ds(dc * L, L)] for dc in range(4)]
    wv_regs = [wvbuf[pl.ds(dc * L, L)] for dc in range(4)]

    @plsc.parallel_loop(0, F)
    def _fold(f):
        for dc in range(4):
            o = f * D + dc * L
            basebuf[pl.ds(o, L)] = basebuf[pl.ds(o, L)] + bias_regs[dc]

    xbufs = (xb0, xb1)
    obufs = (ob0, ob1)
    sxs = (sx0, sx1)
    sos = (so0, so1)

    pltpu.async_copy(x_hbm.at[pl.ds(x0, XW)], xb0.at[pl.ds(8, XW)], sx0)
    pltpu.async_copy(
        x_hbm.at[pl.ds(x0 + XW, XW)], xb1.at[pl.ds(8, XW)], sx1)

    @pl.loop(0, BPW, step=2 * CBB)
    def _pair(j):
        for k in range(2):
            n0 = j + k * CBB
            xb, sx = xbufs[k], sxs[k]
            ob, so = obufs[k], sos[k]
            pltpu.make_async_copy(
                x_hbm.at[pl.ds(x0, XW)], xb.at[pl.ds(8, XW)], sx).wait()

            @pl.when(n0 >= 2 * CBB)
            def _():
                pltpu.make_async_copy(
                    ob, out_hbm.at[pl.ds(b0, CBB)], so).wait()

            @plsc.parallel_loop(0, F, unroll=2)
            def _row(f):
                base_f = [basebuf[pl.ds(f * D + dc * L, L)]
                          for dc in range(4)]
                for bi in range(CBB):
                    xs = plsc.load_gather(
                        xb, [jnp.full((L,), 8 + bi * F + f, jnp.int32)])
                    sub = ob.at[bi]
                    for dc in range(4):
                        sub[f, pl.ds(dc * L, L)] = (
                            xs * wv_regs[dc] + base_f[dc])

            pltpu.async_copy(ob, out_hbm.at[pl.ds(b0 + n0, CBB)], so)

            @pl.when(n0 + 2 * CBB < BPW)
            def _():
                pltpu.async_copy(
                    x_hbm.at[pl.ds(x0 + (n0 + 2 * CBB) * F, XW)],
                    xb.at[pl.ds(8, XW)], sx)

    pltpu.make_async_copy(obuf := obufs[0], out_hbm.at[pl.ds(b0, CBB)],
                          sos[0]).wait()
    pltpu.make_async_copy(obufs[1], out_hbm.at[pl.ds(b0, CBB)],
                          sos[1]).wait()


@jax.jit
def kernel(x, emb_table, W, b):
    mesh = plsc.VectorSubcoreMesh(
        core_axis_name="c", subcore_axis_name="s",
        num_cores=NC, num_subcores=NS)
    return pl.kernel(
        _sc_body,
        out_type=jax.ShapeDtypeStruct((B, F, D), jnp.float32),
        mesh=mesh,
        scratch_types=(
            [
                pltpu.VMEM((F * D,), jnp.float32),
                pltpu.VMEM((D,), jnp.float32),
                pltpu.VMEM((D,), jnp.float32),
            ]
            + [pltpu.VMEM((8 + XW,), jnp.float32) for _ in range(2)]
            + [pltpu.VMEM((CBB, F, D), jnp.float32) for _ in range(2)]
            + [pltpu.SemaphoreType.DMA for _ in range(4)]
        ),
        compiler_params=pltpu.CompilerParams(needs_layout_passes=False),
    )(x.reshape(-1), emb_table.reshape(-1), W.reshape(-1), b)


# speedup vs baseline: 1.8745x; 1.0049x over previous
import jax
import jax.numpy as jnp
from jax import lax
from jax.experimental import pallas as pl
from jax.experimental.pallas import tpu as pltpu
from jax.experimental.pallas import tpu_sc as plsc

B, F, D = 16384, 100, 64
NC, NS, L = 2, 16, 16
NW = NC * NS
BPW = B // NW
CBB = 4
XW = CBB * F


def _sc_body(x_hbm, emb_hbm, w_hbm, bias_hbm, out_hbm,
             basebuf, wvbuf, biasbuf,
             xb0, xb1, ob0, ob1,
             sx0, sx1, so0, so1):
    wid = lax.axis_index("s") * NC + lax.axis_index("c")
    x0 = wid * (BPW * F)
    b0 = wid * BPW

    pltpu.sync_copy(emb_hbm, basebuf)
    pltpu.sync_copy(w_hbm, wvbuf)
    pltpu.sync_copy(bias_hbm, biasbuf)

    bias_regs = [biasbuf[pl.ds(dc * L, L)] for dc in range(4)]
    wv_regs = [wvbuf[pl.ds(dc * L, L)] for dc in range(4)]

    @plsc.parallel_loop(0, F)
    def _fold(f):
        for dc in range(4):
            o = f * D + dc * L
            basebuf[pl.ds(o, L)] = basebuf[pl.ds(o, L)] + bias_regs[dc]

    xbufs = (xb0, xb1)
    obufs = (ob0, ob1)
    sxs = (sx0, sx1)
    sos = (so0, so1)

    pltpu.async_copy(x_hbm.at[pl.ds(x0, XW)], xb0.at[pl.ds(8, XW)], sx0)
    pltpu.async_copy(
        x_hbm.at[pl.ds(x0 + XW, XW)], xb1.at[pl.ds(8, XW)], sx1)

    @pl.loop(0, BPW, step=2 * CBB)
    def _pair(j):
        for k in range(2):
            n0 = j + k * CBB
            xb, sx = xbufs[k], sxs[k]
            ob, so = obufs[k], sos[k]
            pltpu.make_async_copy(
                x_hbm.at[pl.ds(x0, XW)], xb.at[pl.ds(8, XW)], sx).wait()

            @pl.when(n0 >= 2 * CBB)
            def _():
                pltpu.make_async_copy(
                    ob, out_hbm.at[pl.ds(b0, CBB)], so).wait()

            @plsc.parallel_loop(0, F, unroll=2)
            def _row(f):
                base_f = [basebuf[pl.ds(f * D + dc * L, L)]
                          for dc in range(4)]
                for bi in range(CBB):
                    xs = plsc.load_gather(
                        xb, [jnp.full((L,), 8 + bi * F + f, jnp.int32)])
                    sub = ob.at[bi]
                    for dc in range(4):
                        sub[f, pl.ds(dc * L, L)] = (
                            xs * wv_regs[dc] + base_f[dc])

            pltpu.async_copy(ob, out_hbm.at[pl.ds(b0 + n0, CBB)], so)

            @pl.when(n0 + 2 * CBB < BPW)
            def _():
                pltpu.async_copy(
                    x_hbm.at[pl.ds(x0 + (n0 + 2 * CBB) * F, XW)],
                    xb.at[pl.ds(8, XW)], sx)

    for k in range(2):
        pltpu.make_async_copy(
            obufs[k], out_hbm.at[pl.ds(b0, CBB)], sos[k]).wait()


@jax.jit
def kernel(x, emb_table, W, b):
    mesh = plsc.VectorSubcoreMesh(
        core_axis_name="c", subcore_axis_name="s",
        num_cores=NC, num_subcores=NS)
    return pl.kernel(
        _sc_body,
        out_type=jax.ShapeDtypeStruct((B, F, D), jnp.float32),
        mesh=mesh,
        scratch_types=(
            [
                pltpu.VMEM((F * D,), jnp.float32),
                pltpu.VMEM((D,), jnp.float32),
                pltpu.VMEM((D,), jnp.float32),
            ]
            + [pltpu.VMEM((8 + XW,), jnp.float32) for _ in range(2)]
            + [pltpu.VMEM((CBB, F, D), jnp.float32) for _ in range(2)]
            + [pltpu.SemaphoreType.DMA for _ in range(4)]
        ),
        compiler_params=pltpu.CompilerParams(needs_layout_passes=False),
    )(x.reshape(-1), emb_table.reshape(-1), W.reshape(-1), b)
